# Initial kernel scaffold; baseline (speedup 1.0000x reference)
import jax
import jax.numpy as jnp
from jax.experimental import pallas as pl


def kernel(UNITS, UNITS_COUNT, RESOURCES, GLOBAL, input_mask, emb_UNITS, conv_w, conv_b):
    raise NotImplementedError("write your pallas kernel here")


# speedup vs baseline: 8.6344x; 8.6344x over previous
import jax
import jax.numpy as jnp
from jax.experimental import pallas as pl


def _fused_step(u_ref, c_ref, res_ref, g_ref, m_ref, wt_ref, b_ref, e1_ref,
                out_ref):
    m = m_ref[0]
    res = res_ref[0]
    g = g_ref[0]
    hw = res.shape[1]
    gb = jnp.broadcast_to(g, (g.shape[0], hw))
    cont = jnp.concatenate([res, gb], axis=0) * m
    conv = jnp.dot(wt_ref[:], cont,
                   preferred_element_type=jnp.float32)
    conv = conv + b_ref[:]

    u = u_ref[0].astype(jnp.float32)
    s = u * c_ref[0] * m
    e1 = e1_ref[:]
    ea = e1 * s[0:1, :]
    eb = e1 * s[1:2, :]
    out_ref[0, 0] = conv + jnp.concatenate([ea, eb], axis=0)
    out_ref[0, 1] = conv + jnp.concatenate([eb, ea], axis=0)


def kernel(UNITS, UNITS_COUNT, RESOURCES, GLOBAL, input_mask, emb_UNITS,
           conv_w, conv_b):
    B = UNITS.shape[0]
    H, W = UNITS.shape[3], UNITS.shape[4]
    HW = H * W
    D = conv_w.shape[1]
    DPER = emb_UNITS.shape[1]

    u = UNITS.reshape(B, 2, HW)
    c = UNITS_COUNT.reshape(B, 2, HW)
    res = RESOURCES.reshape(B, RESOURCES.shape[1], HW)
    g = GLOBAL.reshape(B, GLOBAL.shape[1], 1)
    m = input_mask.reshape(B, 1, HW)
    wt = conv_w.T
    bcol = conv_b.reshape(D, 1)
    e1col = emb_UNITS[1].reshape(DPER, 1)

    out = pl.pallas_call(
        _fused_step,
        grid=(B,),
        in_specs=[
            pl.BlockSpec((1, 2, HW), lambda i: (i, 0, 0)),
            pl.BlockSpec((1, 2, HW), lambda i: (i, 0, 0)),
            pl.BlockSpec((1, res.shape[1], HW), lambda i: (i, 0, 0)),
            pl.BlockSpec((1, g.shape[1], 1), lambda i: (i, 0, 0)),
            pl.BlockSpec((1, 1, HW), lambda i: (i, 0, 0)),
            pl.BlockSpec((D, wt.shape[1]), lambda i: (0, 0)),
            pl.BlockSpec((D, 1), lambda i: (0, 0)),
            pl.BlockSpec((DPER, 1), lambda i: (0, 0)),
        ],
        out_specs=pl.BlockSpec((1, 2, D, HW), lambda i: (i, 0, 0, 0)),
        out_shape=jax.ShapeDtypeStruct((B, 2, D, HW), jnp.float32),
    )(u, c, res, g, m, wt, bcol, e1col)

    out = out.reshape(2 * B, D, H, W)
    mask2 = jnp.broadcast_to(input_mask[:, None], (B, 2, 1, H, W))
    mask2 = mask2.reshape(2 * B, 1, H, W)
    return (out, mask2)
